# Initial kernel scaffold; baseline (speedup 1.0000x reference)
import numpy as np
import jax
import jax.numpy as jnp
from jax import lax
from jax.experimental import pallas as pl

_LOG_K = float(np.log(64))


def _body(u_ref, vf_ref, bf_ref, ts_ref, coef_ref, vp_ref, lnvt_ref, lv0_ref):
    R = u_ref.shape[0]
    u = u_ref[...]
    vf = vf_ref[...]
    bf = bf_ref[...]
    ts = ts_ref[...]
    coef = coef_ref[...]

    iota_t = lax.broadcasted_iota(jnp.float32, (64, 128), 1)
    tsoh = (iota_t == ts).astype(jnp.float32)
    lal1_b = jnp.dot(tsoh, coef, precision=lax.Precision.HIGHEST,
                     preferred_element_type=jnp.float32)

    iota_c = lax.broadcasted_iota(jnp.float32, (R, 64), 1)
    boh = (iota_c == bf).astype(jnp.float32)
    lal1 = jnp.dot(boh, lal1_b, precision=lax.Precision.HIGHEST,
                   preferred_element_type=jnp.float32)
    la = lal1[:, 0:1]
    l1ma = lal1[:, 1:2]

    oh_v = (iota_c == vf).astype(jnp.float32)
    log_v0 = jnp.log(jnp.clip(oh_v, 1e-30, 1.0))

    a = log_v0 + la
    b = l1ma - _LOG_K
    m = jnp.maximum(a, b)
    log_q = m + jnp.log(jnp.exp(a - m) + jnp.exp(b - m))

    gumbel = -jnp.log(-jnp.log(u + 1e-30) + 1e-30)
    s = gumbel + log_q
    smax = jnp.max(s, axis=1, keepdims=True)
    widx = jnp.min(jnp.where(s == smax, iota_c, 64.0), axis=1, keepdims=True)

    vp = (iota_c == widx).astype(jnp.float32)
    vp_ref[...] = vp
    lnvt_ref[...] = jnp.log(jnp.clip(vp, 1e-30, 1.0))
    lv0_ref[...] = log_v0


def kernel(v, time_step, batch, u, log_alphas_bar, log_1_min_alphas_bar):
    N, C = u.shape
    R = 1024
    G = N // R
    vf = v.astype(jnp.float32).reshape(N, 1)
    bf = batch.astype(jnp.float32).reshape(N, 1)
    tsf = time_step.astype(jnp.float32).reshape(-1, 1)
    T = log_alphas_bar.shape[0]
    coef = jnp.zeros((128, 2), jnp.float32)
    coef = coef.at[:T, 0].set(log_alphas_bar).at[:T, 1].set(log_1_min_alphas_bar)

    row_spec = pl.BlockSpec((R, C), lambda i: (i, 0))
    col_spec = pl.BlockSpec((R, 1), lambda i: (i, 0))
    ts_spec = pl.BlockSpec((64, 1), lambda i: (0, 0))
    coef_spec = pl.BlockSpec((128, 2), lambda i: (0, 0))
    out_sds = jax.ShapeDtypeStruct((N, C), jnp.float32)

    vp, lnvt, lv0 = pl.pallas_call(
        _body,
        grid=(G,),
        in_specs=[row_spec, col_spec, col_spec, ts_spec, coef_spec],
        out_specs=[row_spec, row_spec, row_spec],
        out_shape=[out_sds, out_sds, out_sds],
    )(u, vf, bf, tsf, coef)
    return (vp, lnvt, lv0)


# speedup vs baseline: 4.3262x; 4.3262x over previous
import numpy as np
import jax
import jax.numpy as jnp
from jax import lax
from jax.experimental import pallas as pl

_LOG_K = float(np.log(64))


def _body(u_ref, vf_ref, bf_ref, ts_ref, coef_ref, vp_ref, lnvt_ref, lv0_ref):
    R = u_ref.shape[0]
    u = u_ref[...]
    vf = vf_ref[...]
    bf = bf_ref[...]
    ts = ts_ref[...]
    coef = coef_ref[...]

    iota_t = lax.broadcasted_iota(jnp.int32, (64, 128), 1).astype(jnp.float32)
    tsoh = (iota_t == ts).astype(jnp.float32)
    lal1_b = jnp.dot(tsoh, coef, precision=lax.Precision.HIGHEST,
                     preferred_element_type=jnp.float32)

    iota_c = lax.broadcasted_iota(jnp.int32, (R, 64), 1).astype(jnp.float32)
    boh = (iota_c == bf).astype(jnp.float32)
    lal1 = jnp.dot(boh, lal1_b, precision=lax.Precision.HIGHEST,
                   preferred_element_type=jnp.float32)
    la = lal1[:, 0:1]
    l1ma = lal1[:, 1:2]

    oh_v = (iota_c == vf).astype(jnp.float32)
    log_v0 = jnp.log(jnp.clip(oh_v, 1e-30, 1.0))

    a = log_v0 + la
    b = l1ma - _LOG_K
    m = jnp.maximum(a, b)
    log_q = m + jnp.log(jnp.exp(a - m) + jnp.exp(b - m))

    gumbel = -jnp.log(-jnp.log(u + 1e-30) + 1e-30)
    s = gumbel + log_q
    smax = jnp.max(s, axis=1, keepdims=True)
    widx = jnp.min(jnp.where(s == smax, iota_c, 64.0), axis=1, keepdims=True)

    vp = (iota_c == widx).astype(jnp.float32)
    vp_ref[...] = vp
    lnvt_ref[...] = jnp.log(jnp.clip(vp, 1e-30, 1.0))
    lv0_ref[...] = log_v0


def kernel(v, time_step, batch, u, log_alphas_bar, log_1_min_alphas_bar):
    N, C = u.shape
    R = 1024
    G = N // R
    vf = v.astype(jnp.float32).reshape(N, 1)
    bf = batch.astype(jnp.float32).reshape(N, 1)
    tsf = time_step.astype(jnp.float32).reshape(-1, 1)
    T = log_alphas_bar.shape[0]
    coef = jnp.zeros((128, 2), jnp.float32)
    coef = coef.at[:T, 0].set(log_alphas_bar).at[:T, 1].set(log_1_min_alphas_bar)

    row_spec = pl.BlockSpec((R, C), lambda i: (i, 0))
    col_spec = pl.BlockSpec((R, 1), lambda i: (i, 0))
    ts_spec = pl.BlockSpec((64, 1), lambda i: (0, 0))
    coef_spec = pl.BlockSpec((128, 2), lambda i: (0, 0))
    out_sds = jax.ShapeDtypeStruct((N, C), jnp.float32)

    vp, lnvt, lv0 = pl.pallas_call(
        _body,
        grid=(G,),
        in_specs=[row_spec, col_spec, col_spec, ts_spec, coef_spec],
        out_specs=[row_spec, row_spec, row_spec],
        out_shape=[out_sds, out_sds, out_sds],
    )(u, vf, bf, tsf, coef)
    return (vp, lnvt, lv0)
